# baseline (speedup 1.0000x reference)
import functools

import jax
import jax.numpy as jnp
from jax import lax
from jax.experimental import pallas as pl
from jax.experimental.pallas import tpu as pltpu
from jax.experimental.pallas import tpu_sc as plsc

N = 10000
E = 10000
NODE_F = 32
EDGE_F = 16
HID = 90
NGRAPH = 64

NP = 10240
EP = 10240
NC = 2
NS = 16
NW = NC * NS
EW = EP // NW
CH = 80
NCH = EW // CH
ROWS_SUB = NP // NS

BE = 256
NE = EP // BE
BN = 512
NN = NP // BN

F = 128
HP = 96
HP2 = 48


def _sc_mesh():
    return plsc.VectorSubcoreMesh(core_axis_name="c", subcore_axis_name="s")


def _gather_rows(table, idx):
    @functools.partial(
        pl.kernel,
        out_type=jax.ShapeDtypeStruct((EP, F), jnp.float32),
        mesh=_sc_mesh(),
        scratch_types=[
            pltpu.VMEM((EW,), jnp.int32),
            pltpu.VMEM((EW, F), jnp.float32),
            pltpu.SemaphoreType.DMA,
        ],
    )
    def k(table_hbm, idx_hbm, out_hbm, idx_v, rows_v, sem):
        c = lax.axis_index("c")
        s = lax.axis_index("s")
        base = (s * NC + c) * EW
        pltpu.sync_copy(idx_hbm.at[pl.ds(base, EW)], idx_v)
        cps = [
            pltpu.async_copy(
                table_hbm.at[idx_v.at[pl.ds(j * CH, CH)]],
                rows_v.at[pl.ds(j * CH, CH)],
                sem,
            )
            for j in range(NCH)
        ]
        for cp in cps:
            cp.wait()
        pltpu.sync_copy(rows_v, out_hbm.at[pl.ds(base, EW)])

    return k(table, idx)


def _scatter_add(msg, dst3, zeros):
    @functools.partial(
        pl.kernel,
        out_type=jax.ShapeDtypeStruct((NC, NP, F), jnp.float32),
        mesh=_sc_mesh(),
        scratch_types=[
            pltpu.VMEM((NCH, CH), jnp.int32),
            pltpu.VMEM((EW, F), jnp.float32),
            pltpu.VMEM_SHARED((NP, F), jnp.float32),
            pltpu.SemaphoreType.DMA,
            pltpu.SemaphoreType.DMA,
        ],
    )
    def k(msg_hbm, dst_hbm, zeros_hbm, out_hbm, idx_v, rows_v, acc_sh, zsem, sem):
        c = lax.axis_index("c")
        s = lax.axis_index("s")
        wid = s * NC + c
        base = wid * EW
        zcp = pltpu.async_copy(
            zeros_hbm, acc_sh.at[pl.ds(s * ROWS_SUB, ROWS_SUB)], zsem)
        pltpu.sync_copy(dst_hbm.at[wid], idx_v)
        pltpu.sync_copy(msg_hbm.at[pl.ds(base, EW)], rows_v)
        zcp.wait()
        plsc.subcore_barrier()
        cps = [
            pltpu.async_copy(
                rows_v.at[pl.ds(j * CH, CH)],
                acc_sh.at[idx_v.at[j]],
                sem,
                add=True,
            )
            for j in range(NCH)
        ]
        for cp in cps:
            cp.wait()
        plsc.subcore_barrier()
        pltpu.sync_copy(
            acc_sh.at[pl.ds(s * ROWS_SUB, ROWS_SUB)],
            out_hbm.at[c, pl.ds(s * ROWS_SUB, ROWS_SUB)],
        )

    return k(msg, dst3, zeros)


def _edge_messages(eaT, xj, w1T, b1c, w2pT, b2T, nin, op):
    ng = nin // 2

    def body(eaT_ref, xj_ref, w1T_ref, b1c_ref, w2pT_ref, b2T_ref, out_ref):
        hT = jnp.maximum(
            jnp.dot(w1T_ref[...], eaT_ref[...]) + b1c_ref[...], 0.0)
        hTb = hT.astype(jnp.bfloat16)
        xT = jnp.transpose(xj_ref[...])
        xTr = xT.astype(jnp.bfloat16).astype(jnp.float32)
        accT = jnp.dot(b2T_ref[...], xT)
        for g in range(ng):
            ctg = jnp.dot(w2pT_ref[pl.ds(g * 2 * op, 2 * op), :], hTb,
                          preferred_element_type=jnp.float32)
            ctg = ctg.astype(jnp.bfloat16).astype(jnp.float32)
            for j in range(2):
                i = 2 * g + j
                accT = accT + xTr[i:i + 1, :] * ctg[j * op:(j + 1) * op, :]
        accT = jnp.concatenate(
            [accT, jnp.zeros((F - op, BE), jnp.float32)], axis=0)
        out_ref[...] = jnp.transpose(accT)

    return pl.pallas_call(
        body,
        grid=(NE,),
        in_specs=[
            pl.BlockSpec((EDGE_F, BE), lambda e: (0, e)),
            pl.BlockSpec((BE, F), lambda e: (e, 0)),
            pl.BlockSpec((HID, EDGE_F), lambda e: (0, 0)),
            pl.BlockSpec((HID, 1), lambda e: (0, 0)),
            pl.BlockSpec((nin * op, HID), lambda e: (0, 0)),
            pl.BlockSpec((op, F), lambda e: (0, 0)),
        ],
        out_specs=pl.BlockSpec((BE, F), lambda e: (e, 0)),
        out_shape=jax.ShapeDtypeStruct((EP, F), jnp.float32),
    )(eaT, xj, w1T, b1c, w2pT, b2T)


def _node_update(xl, a0, a1, rootp, biasp):
    def body(x_ref, a0_ref, a1_ref, r_ref, b_ref, o_ref):
        v = jnp.dot(x_ref[...], r_ref[...]) + a0_ref[...] + a1_ref[...] + b_ref[...]
        o_ref[...] = jnp.maximum(v, 0.0)

    return pl.pallas_call(
        body,
        grid=(NN,),
        in_specs=[
            pl.BlockSpec((BN, F), lambda i: (i, 0)),
            pl.BlockSpec((BN, F), lambda i: (i, 0)),
            pl.BlockSpec((BN, F), lambda i: (i, 0)),
            pl.BlockSpec((F, F), lambda i: (0, 0)),
            pl.BlockSpec((1, F), lambda i: (0, 0)),
        ],
        out_specs=pl.BlockSpec((BN, F), lambda i: (i, 0)),
        out_shape=jax.ShapeDtypeStruct((NP, F), jnp.float32),
    )(xl, a0, a1, rootp, biasp)


def _pool_head(batch_row, h, fc1p, fc1bp, outwp, outbp):
    def body(b_ref, h_ref, w_ref, wb_ref, ow_ref, ob_ref, o_ref, g_ref):
        i = pl.program_id(0)
        ids = b_ref[...]
        rows = lax.broadcasted_iota(jnp.int32, (NGRAPH, BN), 0)
        oh = (rows == ids).astype(jnp.float32)
        contrib = jnp.dot(oh, h_ref[...], precision=lax.Precision.HIGHEST)

        @pl.when(i == 0)
        def _():
            g_ref[...] = contrib

        @pl.when(i > 0)
        def _():
            g_ref[...] = g_ref[...] + contrib

        @pl.when(i == NN - 1)
        def _():
            t = jnp.maximum(
                jnp.dot(g_ref[...], w_ref[...]) + wb_ref[...], 0.0)
            o_ref[...] = jnp.dot(t, ow_ref[...]) + ob_ref[...]

    return pl.pallas_call(
        body,
        grid=(NN,),
        in_specs=[
            pl.BlockSpec((1, BN), lambda i: (0, i)),
            pl.BlockSpec((BN, F), lambda i: (i, 0)),
            pl.BlockSpec((F, HP), lambda i: (0, 0)),
            pl.BlockSpec((1, HP), lambda i: (0, 0)),
            pl.BlockSpec((HP, F), lambda i: (0, 0)),
            pl.BlockSpec((1, F), lambda i: (0, 0)),
        ],
        out_specs=pl.BlockSpec((NGRAPH, F), lambda i: (0, 0)),
        out_shape=jax.ShapeDtypeStruct((NGRAPH, F), jnp.float32),
        scratch_shapes=[pltpu.VMEM((NGRAPH, F), jnp.float32)],
    )(batch_row, h, fc1p, fc1bp, outwp, outbp)


def _pad2(a, r, c):
    return jnp.pad(a, ((0, r - a.shape[0]), (0, c - a.shape[1])))


def _prep_w2(w2, b2, nin, nout, op):
    w = w2.reshape(HID, nin, nout)
    w = jnp.pad(w, ((0, 0), (0, 0), (0, op - nout)))
    w2pT = w.transpose(1, 2, 0).reshape(nin * op, HID)
    b = b2.reshape(nin, nout)
    b2T = _pad2(b.T, op, F)
    return w2pT.astype(jnp.bfloat16), b2T


def kernel(x, pos, edge_index, edge_attr, batch,
           mlp1_w1, mlp1_b1, mlp1_w2, mlp1_b2, root1, bias1,
           mlp2_w1, mlp2_b1, mlp2_w2, mlp2_b2, root2, bias2,
           mlp3_w1, mlp3_b1, mlp3_w2, mlp3_b2, root3, bias3,
           fc1_w, fc1_b, out_w, out_b):
    f32 = jnp.float32
    xl = _pad2(jnp.concatenate([x, pos], axis=1), NP, F).astype(f32)

    src = jnp.concatenate([edge_index[0], jnp.zeros((EP - E,), jnp.int32)])
    dst3 = jnp.concatenate([edge_index[1],
                            jnp.full((EP - E,), NP - 1, jnp.int32)]
                           ).reshape(NW, NCH, CH)
    eaT = _pad2(edge_attr, EP, EDGE_F).astype(f32).T
    zeros = jnp.zeros((ROWS_SUB, F), f32)
    batch_row = jnp.concatenate(
        [batch, jnp.full((NP - N,), NGRAPH, jnp.int32)]).reshape(1, NP)

    layers = [
        (NODE_F, HID, HP, mlp1_w1, mlp1_b1, mlp1_w2, mlp1_b2, root1, bias1),
        (HID, HID, HP, mlp2_w1, mlp2_b1, mlp2_w2, mlp2_b2, root2, bias2),
        (HID, HID // 2, HP2, mlp3_w1, mlp3_b1, mlp3_w2, mlp3_b2, root3, bias3),
    ]
    for nin, nout, op, w1, b1, w2, b2, root, bias in layers:
        w2pT, b2T = _prep_w2(w2, b2, nin, nout, op)
        rootp = _pad2(root, F, F)
        biasp = _pad2(bias.reshape(1, nout), 1, F)
        xj = _gather_rows(xl, src)
        msg = _edge_messages(eaT, xj, w1.T, b1.reshape(HID, 1), w2pT, b2T,
                             nin, op)
        aggr = _scatter_add(msg, dst3, zeros)
        xl = _node_update(xl, aggr[0], aggr[1], rootp, biasp)

    fc1p = _pad2(fc1_w, F, HP)
    fc1bp = _pad2(fc1_b.reshape(1, HID), 1, HP)
    outwp = _pad2(out_w, HP, F)
    outbp = _pad2(out_b.reshape(1, 1), 1, F)
    out = _pool_head(batch_row, xl, fc1p, fc1bp, outwp, outbp)
    return out[:, :1]


# speedup vs baseline: 1.0555x; 1.0555x over previous
import functools

import jax
import jax.numpy as jnp
from jax import lax
from jax.experimental import pallas as pl
from jax.experimental.pallas import tpu as pltpu
from jax.experimental.pallas import tpu_sc as plsc

N = 10000
E = 10000
NODE_F = 32
EDGE_F = 16
HID = 90
NGRAPH = 64

NP = 10240
EP = 10240
NC = 2
NS = 16
NW = NC * NS
EW = EP // NW
CH = 80
NCH = EW // CH
ROWS_SUB = NP // NS

BE = 256
NE = EP // BE
BN = 512
NN = NP // BN

F = 128
HP = 96
HP2 = 48


def _sc_mesh():
    return plsc.VectorSubcoreMesh(core_axis_name="c", subcore_axis_name="s")


def _gather_rows(table, idx):
    @functools.partial(
        pl.kernel,
        out_type=jax.ShapeDtypeStruct((EP, F), jnp.float32),
        mesh=_sc_mesh(),
        scratch_types=[
            pltpu.VMEM((EW,), jnp.int32),
            pltpu.VMEM((EW, F), jnp.float32),
            pltpu.SemaphoreType.DMA,
        ],
    )
    def k(table_hbm, idx_hbm, out_hbm, idx_v, rows_v, sem):
        c = lax.axis_index("c")
        s = lax.axis_index("s")
        base = (s * NC + c) * EW
        pltpu.sync_copy(idx_hbm.at[pl.ds(base, EW)], idx_v)
        cps = [
            pltpu.async_copy(
                table_hbm.at[idx_v.at[pl.ds(j * CH, CH)]],
                rows_v.at[pl.ds(j * CH, CH)],
                sem,
            )
            for j in range(NCH)
        ]
        for cp in cps:
            cp.wait()
        pltpu.sync_copy(rows_v, out_hbm.at[pl.ds(base, EW)])

    return k(table, idx)


def _scatter_add(msg, dst3, zeros):
    @functools.partial(
        pl.kernel,
        out_type=jax.ShapeDtypeStruct((NC, NP, F), jnp.float32),
        mesh=_sc_mesh(),
        scratch_types=[
            pltpu.VMEM((NCH, CH), jnp.int32),
            pltpu.VMEM((EW, F), jnp.float32),
            pltpu.VMEM_SHARED((NP, F), jnp.float32),
            pltpu.SemaphoreType.DMA,
            pltpu.SemaphoreType.DMA,
        ],
    )
    def k(msg_hbm, dst_hbm, zeros_hbm, out_hbm, idx_v, rows_v, acc_sh, zsem, sem):
        c = lax.axis_index("c")
        s = lax.axis_index("s")
        wid = s * NC + c
        base = wid * EW
        zcp = pltpu.async_copy(
            zeros_hbm, acc_sh.at[pl.ds(s * ROWS_SUB, ROWS_SUB)], zsem)
        pltpu.sync_copy(dst_hbm.at[wid], idx_v)
        pltpu.sync_copy(msg_hbm.at[pl.ds(base, EW)], rows_v)
        zcp.wait()
        plsc.subcore_barrier()
        cps = [
            pltpu.async_copy(
                rows_v.at[pl.ds(j * CH, CH)],
                acc_sh.at[idx_v.at[j]],
                sem,
                add=True,
            )
            for j in range(NCH)
        ]
        for cp in cps:
            cp.wait()
        plsc.subcore_barrier()
        pltpu.sync_copy(
            acc_sh.at[pl.ds(s * ROWS_SUB, ROWS_SUB)],
            out_hbm.at[c, pl.ds(s * ROWS_SUB, ROWS_SUB)],
        )

    return k(msg, dst3, zeros)


def _edge_messages(eaT, xj, w1T, b1c, w2pT, b2T, nin, op):
    ng = nin // 2

    def body(eaT_ref, xj_ref, w1T_ref, b1c_ref, w2pT_ref, b2T_ref, out_ref):
        hT = jnp.maximum(
            jnp.dot(w1T_ref[...], eaT_ref[...]) + b1c_ref[...], 0.0)
        hTb = hT.astype(jnp.bfloat16)
        xT = jnp.transpose(xj_ref[...])
        xTr = xT.astype(jnp.bfloat16).astype(jnp.float32)
        accT = jnp.dot(b2T_ref[...], xT)
        for g in range(ng):
            ctg = jnp.dot(w2pT_ref[pl.ds(g * 2 * op, 2 * op), :], hTb,
                          preferred_element_type=jnp.float32)
            ctg = ctg.astype(jnp.bfloat16).astype(jnp.float32)
            for j in range(2):
                i = 2 * g + j
                accT = accT + xTr[i:i + 1, :] * ctg[j * op:(j + 1) * op, :]
        accT = jnp.concatenate(
            [accT, jnp.zeros((F - op, BE), jnp.float32)], axis=0)
        out_ref[...] = jnp.transpose(accT)

    return pl.pallas_call(
        body,
        grid=(NE,),
        in_specs=[
            pl.BlockSpec((EDGE_F, BE), lambda e: (0, e)),
            pl.BlockSpec((BE, F), lambda e: (e, 0)),
            pl.BlockSpec((HID, EDGE_F), lambda e: (0, 0)),
            pl.BlockSpec((HID, 1), lambda e: (0, 0)),
            pl.BlockSpec((nin * op, HID), lambda e: (0, 0)),
            pl.BlockSpec((op, F), lambda e: (0, 0)),
        ],
        out_specs=pl.BlockSpec((BE, F), lambda e: (e, 0)),
        out_shape=jax.ShapeDtypeStruct((EP, F), jnp.float32),
    )(eaT, xj, w1T, b1c, w2pT, b2T)


def _node_update(xl, aggr, rootp, biasp):
    def body(x_ref, a0_ref, a1_ref, r_ref, b_ref, o_ref):
        v = (jnp.dot(x_ref[...], r_ref[...])
             + a0_ref[0] + a1_ref[0] + b_ref[...])
        o_ref[...] = jnp.maximum(v, 0.0)

    return pl.pallas_call(
        body,
        grid=(NN,),
        in_specs=[
            pl.BlockSpec((BN, F), lambda i: (i, 0)),
            pl.BlockSpec((1, BN, F), lambda i: (0, i, 0)),
            pl.BlockSpec((1, BN, F), lambda i: (1, i, 0)),
            pl.BlockSpec((F, F), lambda i: (0, 0)),
            pl.BlockSpec((1, F), lambda i: (0, 0)),
        ],
        out_specs=pl.BlockSpec((BN, F), lambda i: (i, 0)),
        out_shape=jax.ShapeDtypeStruct((NP, F), jnp.float32),
    )(xl, aggr, aggr, rootp, biasp)


def _pool_head(batch_row, h, fc1p, fc1bp, outwp, outbp):
    def body(b_ref, h_ref, w_ref, wb_ref, ow_ref, ob_ref, o_ref, g_ref):
        i = pl.program_id(0)
        ids = b_ref[...]
        rows = lax.broadcasted_iota(jnp.int32, (NGRAPH, BN), 0)
        oh = (rows == ids).astype(jnp.float32)
        contrib = jnp.dot(oh, h_ref[...], precision=lax.Precision.HIGHEST)

        @pl.when(i == 0)
        def _():
            g_ref[...] = contrib

        @pl.when(i > 0)
        def _():
            g_ref[...] = g_ref[...] + contrib

        @pl.when(i == NN - 1)
        def _():
            t = jnp.maximum(
                jnp.dot(g_ref[...], w_ref[...]) + wb_ref[...], 0.0)
            o_ref[...] = jnp.dot(t, ow_ref[...]) + ob_ref[...]

    return pl.pallas_call(
        body,
        grid=(NN,),
        in_specs=[
            pl.BlockSpec((1, BN), lambda i: (0, i)),
            pl.BlockSpec((BN, F), lambda i: (i, 0)),
            pl.BlockSpec((F, HP), lambda i: (0, 0)),
            pl.BlockSpec((1, HP), lambda i: (0, 0)),
            pl.BlockSpec((HP, F), lambda i: (0, 0)),
            pl.BlockSpec((1, F), lambda i: (0, 0)),
        ],
        out_specs=pl.BlockSpec((NGRAPH, F), lambda i: (0, 0)),
        out_shape=jax.ShapeDtypeStruct((NGRAPH, F), jnp.float32),
        scratch_shapes=[pltpu.VMEM((NGRAPH, F), jnp.float32)],
    )(batch_row, h, fc1p, fc1bp, outwp, outbp)


def _pad2(a, r, c):
    return jnp.pad(a, ((0, r - a.shape[0]), (0, c - a.shape[1])))


def _prep_w2(w2, b2, nin, nout, op):
    w = w2.reshape(HID, nin, nout)
    w = jnp.pad(w, ((0, 0), (0, 0), (0, op - nout)))
    w2pT = w.transpose(1, 2, 0).reshape(nin * op, HID)
    b = b2.reshape(nin, nout)
    b2T = _pad2(b.T, op, F)
    return w2pT.astype(jnp.bfloat16), b2T


def kernel(x, pos, edge_index, edge_attr, batch,
           mlp1_w1, mlp1_b1, mlp1_w2, mlp1_b2, root1, bias1,
           mlp2_w1, mlp2_b1, mlp2_w2, mlp2_b2, root2, bias2,
           mlp3_w1, mlp3_b1, mlp3_w2, mlp3_b2, root3, bias3,
           fc1_w, fc1_b, out_w, out_b):
    f32 = jnp.float32
    xl = _pad2(jnp.concatenate([x, pos], axis=1), NP, F).astype(f32)

    src = jnp.concatenate([edge_index[0], jnp.zeros((EP - E,), jnp.int32)])
    dst3 = jnp.concatenate([edge_index[1],
                            jnp.full((EP - E,), NP - 1, jnp.int32)]
                           ).reshape(NW, NCH, CH)
    eaT = _pad2(edge_attr, EP, EDGE_F).astype(f32).T
    zeros = jnp.zeros((ROWS_SUB, F), f32)
    batch_row = jnp.concatenate(
        [batch, jnp.full((NP - N,), NGRAPH, jnp.int32)]).reshape(1, NP)

    layers = [
        (NODE_F, HID, HP, mlp1_w1, mlp1_b1, mlp1_w2, mlp1_b2, root1, bias1),
        (HID, HID, HP, mlp2_w1, mlp2_b1, mlp2_w2, mlp2_b2, root2, bias2),
        (HID, HID // 2, HP2, mlp3_w1, mlp3_b1, mlp3_w2, mlp3_b2, root3, bias3),
    ]
    for nin, nout, op, w1, b1, w2, b2, root, bias in layers:
        w2pT, b2T = _prep_w2(w2, b2, nin, nout, op)
        rootp = _pad2(root, F, F)
        biasp = _pad2(bias.reshape(1, nout), 1, F)
        xj = _gather_rows(xl, src)
        msg = _edge_messages(eaT, xj, w1.T, b1.reshape(HID, 1), w2pT, b2T,
                             nin, op)
        aggr = _scatter_add(msg, dst3, zeros)
        xl = _node_update(xl, aggr, rootp, biasp)

    fc1p = _pad2(fc1_w, F, HP)
    fc1bp = _pad2(fc1_b.reshape(1, HID), 1, HP)
    outwp = _pad2(out_w, HP, F)
    outbp = _pad2(out_b.reshape(1, 1), 1, F)
    out = _pool_head(batch_row, xl, fc1p, fc1bp, outwp, outbp)
    return out[:, :1]


# speedup vs baseline: 1.1457x; 1.0855x over previous
import functools

import jax
import jax.numpy as jnp
from jax import lax
from jax.experimental import pallas as pl
from jax.experimental.pallas import tpu as pltpu
from jax.experimental.pallas import tpu_sc as plsc

N = 10000
E = 10000
NODE_F = 32
EDGE_F = 16
HID = 90
NGRAPH = 64

NP = 10240
EP = 10240
NC = 2
NS = 16
NW = NC * NS
EW = EP // NW
CH = 80
NCH = EW // CH
ROWS_SUB = NP // NS

BE = 512
NE = EP // BE
BN = 512
NN = NP // BN

F = 128
HP = 96
HP2 = 48


def _sc_mesh():
    return plsc.VectorSubcoreMesh(core_axis_name="c", subcore_axis_name="s")


def _gather_rows(table, idx):
    @functools.partial(
        pl.kernel,
        out_type=jax.ShapeDtypeStruct((EP, F), jnp.float32),
        mesh=_sc_mesh(),
        scratch_types=[
            pltpu.VMEM((EW,), jnp.int32),
            pltpu.VMEM((EW, F), jnp.float32),
            pltpu.SemaphoreType.DMA,
        ],
    )
    def k(table_hbm, idx_hbm, out_hbm, idx_v, rows_v, sem):
        c = lax.axis_index("c")
        s = lax.axis_index("s")
        base = (s * NC + c) * EW
        pltpu.sync_copy(idx_hbm.at[pl.ds(base, EW)], idx_v)
        cps = [
            pltpu.async_copy(
                table_hbm.at[idx_v.at[pl.ds(j * CH, CH)]],
                rows_v.at[pl.ds(j * CH, CH)],
                sem,
            )
            for j in range(NCH)
        ]
        for cp in cps:
            cp.wait()
        pltpu.sync_copy(rows_v, out_hbm.at[pl.ds(base, EW)])

    return k(table, idx)


def _scatter_add(msg, dst3, zeros):
    @functools.partial(
        pl.kernel,
        out_type=jax.ShapeDtypeStruct((NC, NP, F), jnp.float32),
        mesh=_sc_mesh(),
        scratch_types=[
            pltpu.VMEM((NCH, CH), jnp.int32),
            pltpu.VMEM((EW, F), jnp.float32),
            pltpu.VMEM_SHARED((NP, F), jnp.float32),
            pltpu.SemaphoreType.DMA,
            pltpu.SemaphoreType.DMA,
        ],
    )
    def k(msg_hbm, dst_hbm, zeros_hbm, out_hbm, idx_v, rows_v, acc_sh, zsem, sem):
        c = lax.axis_index("c")
        s = lax.axis_index("s")
        wid = s * NC + c
        base = wid * EW
        zcp = pltpu.async_copy(
            zeros_hbm, acc_sh.at[pl.ds(s * ROWS_SUB, ROWS_SUB)], zsem)
        pltpu.sync_copy(dst_hbm.at[wid], idx_v)
        pltpu.sync_copy(msg_hbm.at[pl.ds(base, EW)], rows_v)
        zcp.wait()
        plsc.subcore_barrier()
        cps = [
            pltpu.async_copy(
                rows_v.at[pl.ds(j * CH, CH)],
                acc_sh.at[idx_v.at[j]],
                sem,
                add=True,
            )
            for j in range(NCH)
        ]
        for cp in cps:
            cp.wait()
        plsc.subcore_barrier()
        pltpu.sync_copy(
            acc_sh.at[pl.ds(s * ROWS_SUB, ROWS_SUB)],
            out_hbm.at[c, pl.ds(s * ROWS_SUB, ROWS_SUB)],
        )

    return k(msg, dst3, zeros)


def _edge_messages(eaT, xj, w1T, b1c, w2pT, b2T, nin, op):
    ng = nin // 2

    def body(eaT_ref, xj_ref, w1T_ref, b1c_ref, w2pT_ref, b2T_ref, out_ref):
        hT = jnp.maximum(
            jnp.dot(w1T_ref[...], eaT_ref[...]) + b1c_ref[...], 0.0)
        hTb = hT.astype(jnp.bfloat16)
        xT = jnp.transpose(xj_ref[...])
        xTr = xT.astype(jnp.bfloat16).astype(jnp.float32)
        accT = jnp.dot(b2T_ref[...], xT)
        for g in range(ng):
            ctg = jnp.dot(w2pT_ref[pl.ds(g * 2 * op, 2 * op), :], hTb,
                          preferred_element_type=jnp.float32)
            ctg = ctg.astype(jnp.bfloat16).astype(jnp.float32)
            for j in range(2):
                i = 2 * g + j
                accT = accT + xTr[i:i + 1, :] * ctg[j * op:(j + 1) * op, :]
        accT = jnp.concatenate(
            [accT, jnp.zeros((F - op, BE), jnp.float32)], axis=0)
        out_ref[...] = jnp.transpose(accT)

    return pl.pallas_call(
        body,
        grid=(NE,),
        in_specs=[
            pl.BlockSpec((EDGE_F, BE), lambda e: (0, e)),
            pl.BlockSpec((BE, F), lambda e: (e, 0)),
            pl.BlockSpec((HID, EDGE_F), lambda e: (0, 0)),
            pl.BlockSpec((HID, 1), lambda e: (0, 0)),
            pl.BlockSpec((nin * op, HID), lambda e: (0, 0)),
            pl.BlockSpec((op, F), lambda e: (0, 0)),
        ],
        out_specs=pl.BlockSpec((BE, F), lambda e: (e, 0)),
        out_shape=jax.ShapeDtypeStruct((EP, F), jnp.float32),
    )(eaT, xj, w1T, b1c, w2pT, b2T)


def _node_update(xl, aggr, rootp, biasp):
    def body(x_ref, a0_ref, a1_ref, r_ref, b_ref, o_ref):
        v = (jnp.dot(x_ref[...], r_ref[...])
             + a0_ref[0] + a1_ref[0] + b_ref[...])
        o_ref[...] = jnp.maximum(v, 0.0)

    return pl.pallas_call(
        body,
        grid=(NN,),
        in_specs=[
            pl.BlockSpec((BN, F), lambda i: (i, 0)),
            pl.BlockSpec((1, BN, F), lambda i: (0, i, 0)),
            pl.BlockSpec((1, BN, F), lambda i: (1, i, 0)),
            pl.BlockSpec((F, F), lambda i: (0, 0)),
            pl.BlockSpec((1, F), lambda i: (0, 0)),
        ],
        out_specs=pl.BlockSpec((BN, F), lambda i: (i, 0)),
        out_shape=jax.ShapeDtypeStruct((NP, F), jnp.float32),
    )(xl, aggr, aggr, rootp, biasp)


def _pool_head(batch_row, h, fc1p, fc1bp, outwp, outbp):
    def body(b_ref, h_ref, w_ref, wb_ref, ow_ref, ob_ref, o_ref, g_ref):
        i = pl.program_id(0)
        ids = b_ref[...]
        rows = lax.broadcasted_iota(jnp.int32, (NGRAPH, BN), 0)
        oh = (rows == ids).astype(jnp.float32)
        contrib = jnp.dot(oh, h_ref[...], precision=lax.Precision.HIGHEST)

        @pl.when(i == 0)
        def _():
            g_ref[...] = contrib

        @pl.when(i > 0)
        def _():
            g_ref[...] = g_ref[...] + contrib

        @pl.when(i == NN - 1)
        def _():
            t = jnp.maximum(
                jnp.dot(g_ref[...], w_ref[...]) + wb_ref[...], 0.0)
            o_ref[...] = jnp.dot(t, ow_ref[...]) + ob_ref[...]

    return pl.pallas_call(
        body,
        grid=(NN,),
        in_specs=[
            pl.BlockSpec((1, BN), lambda i: (0, i)),
            pl.BlockSpec((BN, F), lambda i: (i, 0)),
            pl.BlockSpec((F, HP), lambda i: (0, 0)),
            pl.BlockSpec((1, HP), lambda i: (0, 0)),
            pl.BlockSpec((HP, F), lambda i: (0, 0)),
            pl.BlockSpec((1, F), lambda i: (0, 0)),
        ],
        out_specs=pl.BlockSpec((NGRAPH, F), lambda i: (0, 0)),
        out_shape=jax.ShapeDtypeStruct((NGRAPH, F), jnp.float32),
        scratch_shapes=[pltpu.VMEM((NGRAPH, F), jnp.float32)],
    )(batch_row, h, fc1p, fc1bp, outwp, outbp)


def _pad2(a, r, c):
    return jnp.pad(a, ((0, r - a.shape[0]), (0, c - a.shape[1])))


def _prep_w2(w2, b2, nin, nout, op):
    w = w2.reshape(HID, nin, nout)
    w = jnp.pad(w, ((0, 0), (0, 0), (0, op - nout)))
    w2pT = w.transpose(1, 2, 0).reshape(nin * op, HID)
    b = b2.reshape(nin, nout)
    b2T = _pad2(b.T, op, F)
    return w2pT.astype(jnp.bfloat16), b2T


def kernel(x, pos, edge_index, edge_attr, batch,
           mlp1_w1, mlp1_b1, mlp1_w2, mlp1_b2, root1, bias1,
           mlp2_w1, mlp2_b1, mlp2_w2, mlp2_b2, root2, bias2,
           mlp3_w1, mlp3_b1, mlp3_w2, mlp3_b2, root3, bias3,
           fc1_w, fc1_b, out_w, out_b):
    f32 = jnp.float32
    xl = _pad2(jnp.concatenate([x, pos], axis=1), NP, F).astype(f32)

    src = jnp.concatenate([edge_index[0], jnp.zeros((EP - E,), jnp.int32)])
    dst3 = jnp.concatenate([edge_index[1],
                            jnp.full((EP - E,), NP - 1, jnp.int32)]
                           ).reshape(NW, NCH, CH)
    eaT = _pad2(edge_attr, EP, EDGE_F).astype(f32).T
    zeros = jnp.zeros((ROWS_SUB, F), f32)
    batch_row = jnp.concatenate(
        [batch, jnp.full((NP - N,), NGRAPH, jnp.int32)]).reshape(1, NP)

    layers = [
        (NODE_F, HID, HP, mlp1_w1, mlp1_b1, mlp1_w2, mlp1_b2, root1, bias1),
        (HID, HID, HP, mlp2_w1, mlp2_b1, mlp2_w2, mlp2_b2, root2, bias2),
        (HID, HID // 2, HP2, mlp3_w1, mlp3_b1, mlp3_w2, mlp3_b2, root3, bias3),
    ]
    for nin, nout, op, w1, b1, w2, b2, root, bias in layers:
        w2pT, b2T = _prep_w2(w2, b2, nin, nout, op)
        rootp = _pad2(root, F, F)
        biasp = _pad2(bias.reshape(1, nout), 1, F)
        xj = _gather_rows(xl, src)
        msg = _edge_messages(eaT, xj, w1.T, b1.reshape(HID, 1), w2pT, b2T,
                             nin, op)
        aggr = _scatter_add(msg, dst3, zeros)
        xl = _node_update(xl, aggr, rootp, biasp)

    fc1p = _pad2(fc1_w, F, HP)
    fc1bp = _pad2(fc1_b.reshape(1, HID), 1, HP)
    outwp = _pad2(out_w, HP, F)
    outbp = _pad2(out_b.reshape(1, 1), 1, F)
    out = _pool_head(batch_row, xl, fc1p, fc1bp, outwp, outbp)
    return out[:, :1]


# speedup vs baseline: 1.1928x; 1.0411x over previous
import functools

import jax
import jax.numpy as jnp
from jax import lax
from jax.experimental import pallas as pl
from jax.experimental.pallas import tpu as pltpu
from jax.experimental.pallas import tpu_sc as plsc

N = 10000
E = 10000
NODE_F = 32
EDGE_F = 16
HID = 90
NGRAPH = 64

NP = 10240
EP = 10240
NC = 2
NS = 16
NW = NC * NS
EW = EP // NW
CH = 80
NCH = EW // CH
ROWS_SUB = NP // NS

BE = 512
NE = EP // BE
BN = 512
NN = NP // BN

F = 128
HP = 96
HP2 = 48


def _sc_mesh():
    return plsc.VectorSubcoreMesh(core_axis_name="c", subcore_axis_name="s")


def _gather_rows(table, idx):
    @functools.partial(
        pl.kernel,
        out_type=jax.ShapeDtypeStruct((EP, F), jnp.float32),
        mesh=_sc_mesh(),
        scratch_types=[
            pltpu.VMEM((EW,), jnp.int32),
            pltpu.VMEM((EW, F), jnp.float32),
            pltpu.SemaphoreType.DMA,
        ],
    )
    def k(table_hbm, idx_hbm, out_hbm, idx_v, rows_v, sem):
        c = lax.axis_index("c")
        s = lax.axis_index("s")
        base = (s * NC + c) * EW
        pltpu.sync_copy(idx_hbm.at[pl.ds(base, EW)], idx_v)
        cps = [
            pltpu.async_copy(
                table_hbm.at[idx_v.at[pl.ds(j * CH, CH)]],
                rows_v.at[pl.ds(j * CH, CH)],
                sem,
            )
            for j in range(NCH)
        ]
        for cp in cps:
            cp.wait()
        pltpu.sync_copy(rows_v, out_hbm.at[pl.ds(base, EW)])

    return k(table, idx)


def _scatter_add(msg, dst3, zeros):
    @functools.partial(
        pl.kernel,
        out_type=jax.ShapeDtypeStruct((NC, NP, F), jnp.float32),
        mesh=_sc_mesh(),
        scratch_types=[
            pltpu.VMEM((NCH, CH), jnp.int32),
            pltpu.VMEM((EW, F), jnp.float32),
            pltpu.VMEM_SHARED((NP, F), jnp.float32),
            pltpu.SemaphoreType.DMA,
            pltpu.SemaphoreType.DMA,
        ],
    )
    def k(msg_hbm, dst_hbm, zeros_hbm, out_hbm, idx_v, rows_v, acc_sh, zsem, sem):
        c = lax.axis_index("c")
        s = lax.axis_index("s")
        wid = s * NC + c
        base = wid * EW
        zcp = pltpu.async_copy(
            zeros_hbm, acc_sh.at[pl.ds(s * ROWS_SUB, ROWS_SUB)], zsem)
        pltpu.sync_copy(dst_hbm.at[wid], idx_v)
        pltpu.sync_copy(msg_hbm.at[pl.ds(base, EW)], rows_v)
        zcp.wait()
        plsc.subcore_barrier()
        cps = [
            pltpu.async_copy(
                rows_v.at[pl.ds(j * CH, CH)],
                acc_sh.at[idx_v.at[j]],
                sem,
                add=True,
            )
            for j in range(NCH)
        ]
        for cp in cps:
            cp.wait()
        plsc.subcore_barrier()
        pltpu.sync_copy(
            acc_sh.at[pl.ds(s * ROWS_SUB, ROWS_SUB)],
            out_hbm.at[c, pl.ds(s * ROWS_SUB, ROWS_SUB)],
        )

    return k(msg, dst3, zeros)


def _edge_messages(eaT, xj, w1T, b1c, w2pT, b2T, nin, op):
    ng = nin // 2

    def body(eaT_ref, xj_ref, w1T_ref, b1c_ref, w2pT_ref, b2T_ref, out_ref):
        hT = jnp.maximum(
            jnp.dot(w1T_ref[...], eaT_ref[...]) + b1c_ref[...], 0.0)
        hTb = hT.astype(jnp.bfloat16)
        xT = jnp.transpose(xj_ref[...])
        xTr = xT.astype(jnp.bfloat16).astype(jnp.float32)
        accT = jnp.dot(b2T_ref[...], xT)
        for g in range(ng):
            ctg = jnp.dot(w2pT_ref[pl.ds(g * 2 * op, 2 * op), :], hTb,
                          preferred_element_type=jnp.float32)
            ctg = ctg.astype(jnp.bfloat16).astype(jnp.float32)
            for j in range(2):
                i = 2 * g + j
                accT = accT + xTr[i:i + 1, :] * ctg[j * op:(j + 1) * op, :]
        accT = jnp.concatenate(
            [accT, jnp.zeros((F - op, BE), jnp.float32)], axis=0)
        out_ref[...] = jnp.transpose(accT)

    return pl.pallas_call(
        body,
        grid=(NE,),
        in_specs=[
            pl.BlockSpec((EDGE_F, BE), lambda e: (0, e)),
            pl.BlockSpec((BE, F), lambda e: (e, 0)),
            pl.BlockSpec((HID, EDGE_F), lambda e: (0, 0)),
            pl.BlockSpec((HID, 1), lambda e: (0, 0)),
            pl.BlockSpec((nin * op, HID), lambda e: (0, 0)),
            pl.BlockSpec((op, F), lambda e: (0, 0)),
        ],
        out_specs=pl.BlockSpec((BE, F), lambda e: (e, 0)),
        out_shape=jax.ShapeDtypeStruct((EP, F), jnp.float32),
    )(eaT, xj, w1T, b1c, w2pT, b2T)


def _node_update(xl, aggr, rootp, biasp):
    def body(x_ref, a0_ref, a1_ref, r_ref, b_ref, o_ref):
        v = (jnp.dot(x_ref[...], r_ref[...])
             + a0_ref[0] + a1_ref[0] + b_ref[...])
        o_ref[...] = jnp.maximum(v, 0.0)

    return pl.pallas_call(
        body,
        grid=(NN,),
        in_specs=[
            pl.BlockSpec((BN, F), lambda i: (i, 0)),
            pl.BlockSpec((1, BN, F), lambda i: (0, i, 0)),
            pl.BlockSpec((1, BN, F), lambda i: (1, i, 0)),
            pl.BlockSpec((F, F), lambda i: (0, 0)),
            pl.BlockSpec((1, F), lambda i: (0, 0)),
        ],
        out_specs=pl.BlockSpec((BN, F), lambda i: (i, 0)),
        out_shape=jax.ShapeDtypeStruct((NP, F), jnp.float32),
    )(xl, aggr, aggr, rootp, biasp)


def _node_pool_head(xl, aggr, rootp, biasp, batch_row, fc1p, fc1bp, outwp, outbp):
    def body(x_ref, a0_ref, a1_ref, r_ref, b_ref, br_ref, w_ref, wb_ref,
             ow_ref, ob_ref, o_ref, g_ref):
        i = pl.program_id(0)
        h = jnp.maximum(
            jnp.dot(x_ref[...], r_ref[...])
            + a0_ref[0] + a1_ref[0] + b_ref[...], 0.0)
        ids = br_ref[...]
        rows = lax.broadcasted_iota(jnp.int32, (NGRAPH, BN), 0)
        oh = (rows == ids).astype(jnp.float32)
        contrib = jnp.dot(oh, h, precision=lax.Precision.HIGHEST)

        @pl.when(i == 0)
        def _():
            g_ref[...] = contrib

        @pl.when(i > 0)
        def _():
            g_ref[...] = g_ref[...] + contrib

        @pl.when(i == NN - 1)
        def _():
            t = jnp.maximum(
                jnp.dot(g_ref[...], w_ref[...]) + wb_ref[...], 0.0)
            o_ref[...] = jnp.dot(t, ow_ref[...]) + ob_ref[...]

    return pl.pallas_call(
        body,
        grid=(NN,),
        in_specs=[
            pl.BlockSpec((BN, F), lambda i: (i, 0)),
            pl.BlockSpec((1, BN, F), lambda i: (0, i, 0)),
            pl.BlockSpec((1, BN, F), lambda i: (1, i, 0)),
            pl.BlockSpec((F, F), lambda i: (0, 0)),
            pl.BlockSpec((1, F), lambda i: (0, 0)),
            pl.BlockSpec((1, BN), lambda i: (0, i)),
            pl.BlockSpec((F, HP), lambda i: (0, 0)),
            pl.BlockSpec((1, HP), lambda i: (0, 0)),
            pl.BlockSpec((HP, F), lambda i: (0, 0)),
            pl.BlockSpec((1, F), lambda i: (0, 0)),
        ],
        out_specs=pl.BlockSpec((NGRAPH, F), lambda i: (0, 0)),
        out_shape=jax.ShapeDtypeStruct((NGRAPH, F), jnp.float32),
        scratch_shapes=[pltpu.VMEM((NGRAPH, F), jnp.float32)],
    )(xl, aggr, aggr, rootp, biasp, batch_row, fc1p, fc1bp, outwp, outbp)


def _pad2(a, r, c):
    return jnp.pad(a, ((0, r - a.shape[0]), (0, c - a.shape[1])))


def _prep_w2(w2, b2, nin, nout, op):
    w = w2.reshape(HID, nin, nout)
    w = jnp.pad(w, ((0, 0), (0, 0), (0, op - nout)))
    w2pT = w.transpose(1, 2, 0).reshape(nin * op, HID)
    b = b2.reshape(nin, nout)
    b2T = _pad2(b.T, op, F)
    return w2pT.astype(jnp.bfloat16), b2T


def kernel(x, pos, edge_index, edge_attr, batch,
           mlp1_w1, mlp1_b1, mlp1_w2, mlp1_b2, root1, bias1,
           mlp2_w1, mlp2_b1, mlp2_w2, mlp2_b2, root2, bias2,
           mlp3_w1, mlp3_b1, mlp3_w2, mlp3_b2, root3, bias3,
           fc1_w, fc1_b, out_w, out_b):
    f32 = jnp.float32
    xl = _pad2(jnp.concatenate([x, pos], axis=1), NP, F).astype(f32)

    src = jnp.concatenate([edge_index[0], jnp.zeros((EP - E,), jnp.int32)])
    dst3 = jnp.concatenate([edge_index[1],
                            jnp.full((EP - E,), NP - 1, jnp.int32)]
                           ).reshape(NW, NCH, CH)
    eaT = _pad2(edge_attr, EP, EDGE_F).astype(f32).T
    zeros = jnp.zeros((ROWS_SUB, F), f32)
    batch_row = jnp.concatenate(
        [batch, jnp.full((NP - N,), NGRAPH, jnp.int32)]).reshape(1, NP)

    layers = [
        (NODE_F, HID, HP, mlp1_w1, mlp1_b1, mlp1_w2, mlp1_b2, root1, bias1),
        (HID, HID, HP, mlp2_w1, mlp2_b1, mlp2_w2, mlp2_b2, root2, bias2),
        (HID, HID // 2, HP2, mlp3_w1, mlp3_b1, mlp3_w2, mlp3_b2, root3, bias3),
    ]
    aggr = None
    for li, (nin, nout, op, w1, b1, w2, b2, root, bias) in enumerate(layers):
        w2pT, b2T = _prep_w2(w2, b2, nin, nout, op)
        rootp = _pad2(root, F, F)
        biasp = _pad2(bias.reshape(1, nout), 1, F)
        xj = _gather_rows(xl, src)
        msg = _edge_messages(eaT, xj, w1.T, b1.reshape(HID, 1), w2pT, b2T,
                             nin, op)
        aggr = _scatter_add(msg, dst3, zeros)
        if li < 2:
            xl = _node_update(xl, aggr, rootp, biasp)

    fc1p = _pad2(fc1_w, F, HP)
    fc1bp = _pad2(fc1_b.reshape(1, HID), 1, HP)
    outwp = _pad2(out_w, HP, F)
    outbp = _pad2(out_b.reshape(1, 1), 1, F)
    rootp3 = _pad2(root3, F, F)
    biasp3 = _pad2(bias3.reshape(1, HID // 2), 1, F)
    out = _node_pool_head(xl, aggr, rootp3, biasp3, batch_row,
                          fc1p, fc1bp, outwp, outbp)
    return out[:, :1]


# speedup vs baseline: 1.2025x; 1.0082x over previous
import functools

import jax
import jax.numpy as jnp
from jax import lax
from jax.experimental import pallas as pl
from jax.experimental.pallas import tpu as pltpu
from jax.experimental.pallas import tpu_sc as plsc

N = 10000
E = 10000
NODE_F = 32
EDGE_F = 16
HID = 90
NGRAPH = 64

NP = 10240
EP = 10240
NC = 2
NS = 16
NW = NC * NS
EW = EP // NW
CH = 80
NCH = EW // CH
ROWS_SUB = NP // NS

BE = 512
NE = EP // BE
BN = 512
NN = NP // BN

F = 128
HP = 96
HP2 = 48


def _sc_mesh():
    return plsc.VectorSubcoreMesh(core_axis_name="c", subcore_axis_name="s")


def _gather_rows(table, idx):
    @functools.partial(
        pl.kernel,
        out_type=jax.ShapeDtypeStruct((EP, F), jnp.float32),
        mesh=_sc_mesh(),
        scratch_types=[
            pltpu.VMEM((EW,), jnp.int32),
            pltpu.VMEM((EW, F), jnp.float32),
            pltpu.SemaphoreType.DMA,
            pltpu.SemaphoreType.DMA,
        ],
    )
    def k(table_hbm, idx_hbm, out_hbm, idx_v, rows_v, sem, osem):
        c = lax.axis_index("c")
        s = lax.axis_index("s")
        base = (s * NC + c) * EW
        pltpu.sync_copy(idx_hbm.at[pl.ds(base, EW)], idx_v)
        cps = [
            pltpu.async_copy(
                table_hbm.at[idx_v.at[pl.ds(j * CH, CH)]],
                rows_v.at[pl.ds(j * CH, CH)],
                sem,
            )
            for j in range(NCH)
        ]
        ocps = []
        for j, cp in enumerate(cps):
            cp.wait()
            ocps.append(pltpu.async_copy(
                rows_v.at[pl.ds(j * CH, CH)],
                out_hbm.at[pl.ds(base + j * CH, CH)],
                osem,
            ))
        for ocp in ocps:
            ocp.wait()

    return k(table, idx)


def _scatter_add(msg, dst3, zeros):
    @functools.partial(
        pl.kernel,
        out_type=jax.ShapeDtypeStruct((NC, NP, F), jnp.float32),
        mesh=_sc_mesh(),
        scratch_types=[
            pltpu.VMEM((NCH, CH), jnp.int32),
            pltpu.VMEM((EW, F), jnp.float32),
            pltpu.VMEM_SHARED((NP, F), jnp.float32),
            pltpu.SemaphoreType.DMA,
            pltpu.SemaphoreType.DMA,
        ],
    )
    def k(msg_hbm, dst_hbm, zeros_hbm, out_hbm, idx_v, rows_v, acc_sh, zsem, sem):
        c = lax.axis_index("c")
        s = lax.axis_index("s")
        wid = s * NC + c
        base = wid * EW
        zcp = pltpu.async_copy(
            zeros_hbm, acc_sh.at[pl.ds(s * ROWS_SUB, ROWS_SUB)], zsem)
        pltpu.sync_copy(dst_hbm.at[wid], idx_v)
        pltpu.sync_copy(msg_hbm.at[pl.ds(base, EW)], rows_v)
        zcp.wait()
        plsc.subcore_barrier()
        cps = [
            pltpu.async_copy(
                rows_v.at[pl.ds(j * CH, CH)],
                acc_sh.at[idx_v.at[j]],
                sem,
                add=True,
            )
            for j in range(NCH)
        ]
        for cp in cps:
            cp.wait()
        plsc.subcore_barrier()
        pltpu.sync_copy(
            acc_sh.at[pl.ds(s * ROWS_SUB, ROWS_SUB)],
            out_hbm.at[c, pl.ds(s * ROWS_SUB, ROWS_SUB)],
        )

    return k(msg, dst3, zeros)


def _edge_messages(eaT, xj, w1T, b1c, w2pT, b2T, nin, op):
    gw = 384 // op
    while nin % gw:
        gw -= 1
    ng = nin // gw

    def body(eaT_ref, xj_ref, w1T_ref, b1c_ref, w2pT_ref, b2T_ref, out_ref):
        hT = jnp.maximum(
            jnp.dot(w1T_ref[...], eaT_ref[...]) + b1c_ref[...], 0.0)
        hTb = hT.astype(jnp.bfloat16)
        xT = jnp.transpose(xj_ref[...])
        xTr = xT.astype(jnp.bfloat16).astype(jnp.float32)
        accT = jnp.dot(b2T_ref[...], xT)
        for g in range(ng):
            ctg = jnp.dot(w2pT_ref[pl.ds(g * gw * op, gw * op), :], hTb,
                          preferred_element_type=jnp.float32)
            ctg = ctg.astype(jnp.bfloat16).astype(jnp.float32)
            for j in range(gw):
                i = gw * g + j
                accT = accT + xTr[i:i + 1, :] * ctg[j * op:(j + 1) * op, :]
        accT = jnp.concatenate(
            [accT, jnp.zeros((F - op, BE), jnp.float32)], axis=0)
        out_ref[...] = jnp.transpose(accT)

    return pl.pallas_call(
        body,
        grid=(NE,),
        in_specs=[
            pl.BlockSpec((EDGE_F, BE), lambda e: (0, e)),
            pl.BlockSpec((BE, F), lambda e: (e, 0)),
            pl.BlockSpec((HID, EDGE_F), lambda e: (0, 0)),
            pl.BlockSpec((HID, 1), lambda e: (0, 0)),
            pl.BlockSpec((nin * op, HID), lambda e: (0, 0)),
            pl.BlockSpec((op, F), lambda e: (0, 0)),
        ],
        out_specs=pl.BlockSpec((BE, F), lambda e: (e, 0)),
        out_shape=jax.ShapeDtypeStruct((EP, F), jnp.float32),
    )(eaT, xj, w1T, b1c, w2pT, b2T)


def _node_update(xl, aggr, rootp, biasp):
    def body(x_ref, a0_ref, a1_ref, r_ref, b_ref, o_ref):
        v = (jnp.dot(x_ref[...], r_ref[...])
             + a0_ref[0] + a1_ref[0] + b_ref[...])
        o_ref[...] = jnp.maximum(v, 0.0)

    return pl.pallas_call(
        body,
        grid=(NN,),
        in_specs=[
            pl.BlockSpec((BN, F), lambda i: (i, 0)),
            pl.BlockSpec((1, BN, F), lambda i: (0, i, 0)),
            pl.BlockSpec((1, BN, F), lambda i: (1, i, 0)),
            pl.BlockSpec((F, F), lambda i: (0, 0)),
            pl.BlockSpec((1, F), lambda i: (0, 0)),
        ],
        out_specs=pl.BlockSpec((BN, F), lambda i: (i, 0)),
        out_shape=jax.ShapeDtypeStruct((NP, F), jnp.float32),
    )(xl, aggr, aggr, rootp, biasp)


def _node_pool_head(xl, aggr, rootp, biasp, batch_row, fc1p, fc1bp, outwp, outbp):
    def body(x_ref, a0_ref, a1_ref, r_ref, b_ref, br_ref, w_ref, wb_ref,
             ow_ref, ob_ref, o_ref, g_ref):
        i = pl.program_id(0)
        h = jnp.maximum(
            jnp.dot(x_ref[...], r_ref[...])
            + a0_ref[0] + a1_ref[0] + b_ref[...], 0.0)
        ids = br_ref[...]
        rows = lax.broadcasted_iota(jnp.int32, (NGRAPH, BN), 0)
        oh = (rows == ids).astype(jnp.float32)
        contrib = jnp.dot(oh, h, precision=lax.Precision.HIGHEST)

        @pl.when(i == 0)
        def _():
            g_ref[...] = contrib

        @pl.when(i > 0)
        def _():
            g_ref[...] = g_ref[...] + contrib

        @pl.when(i == NN - 1)
        def _():
            t = jnp.maximum(
                jnp.dot(g_ref[...], w_ref[...]) + wb_ref[...], 0.0)
            o_ref[...] = jnp.dot(t, ow_ref[...]) + ob_ref[...]

    return pl.pallas_call(
        body,
        grid=(NN,),
        in_specs=[
            pl.BlockSpec((BN, F), lambda i: (i, 0)),
            pl.BlockSpec((1, BN, F), lambda i: (0, i, 0)),
            pl.BlockSpec((1, BN, F), lambda i: (1, i, 0)),
            pl.BlockSpec((F, F), lambda i: (0, 0)),
            pl.BlockSpec((1, F), lambda i: (0, 0)),
            pl.BlockSpec((1, BN), lambda i: (0, i)),
            pl.BlockSpec((F, HP), lambda i: (0, 0)),
            pl.BlockSpec((1, HP), lambda i: (0, 0)),
            pl.BlockSpec((HP, F), lambda i: (0, 0)),
            pl.BlockSpec((1, F), lambda i: (0, 0)),
        ],
        out_specs=pl.BlockSpec((NGRAPH, F), lambda i: (0, 0)),
        out_shape=jax.ShapeDtypeStruct((NGRAPH, F), jnp.float32),
        scratch_shapes=[pltpu.VMEM((NGRAPH, F), jnp.float32)],
    )(xl, aggr, aggr, rootp, biasp, batch_row, fc1p, fc1bp, outwp, outbp)


def _pad2(a, r, c):
    return jnp.pad(a, ((0, r - a.shape[0]), (0, c - a.shape[1])))


def _prep_w2(w2, b2, nin, nout, op):
    w = w2.reshape(HID, nin, nout)
    w = jnp.pad(w, ((0, 0), (0, 0), (0, op - nout)))
    w2pT = w.transpose(1, 2, 0).reshape(nin * op, HID)
    b = b2.reshape(nin, nout)
    b2T = _pad2(b.T, op, F)
    return w2pT.astype(jnp.bfloat16), b2T


def kernel(x, pos, edge_index, edge_attr, batch,
           mlp1_w1, mlp1_b1, mlp1_w2, mlp1_b2, root1, bias1,
           mlp2_w1, mlp2_b1, mlp2_w2, mlp2_b2, root2, bias2,
           mlp3_w1, mlp3_b1, mlp3_w2, mlp3_b2, root3, bias3,
           fc1_w, fc1_b, out_w, out_b):
    f32 = jnp.float32
    xl = _pad2(jnp.concatenate([x, pos], axis=1), NP, F).astype(f32)

    src = jnp.concatenate([edge_index[0], jnp.zeros((EP - E,), jnp.int32)])
    dst3 = jnp.concatenate([edge_index[1],
                            jnp.full((EP - E,), NP - 1, jnp.int32)]
                           ).reshape(NW, NCH, CH)
    eaT = _pad2(edge_attr, EP, EDGE_F).astype(f32).T
    zeros = jnp.zeros((ROWS_SUB, F), f32)
    batch_row = jnp.concatenate(
        [batch, jnp.full((NP - N,), NGRAPH, jnp.int32)]).reshape(1, NP)

    layers = [
        (NODE_F, HID, HP, mlp1_w1, mlp1_b1, mlp1_w2, mlp1_b2, root1, bias1),
        (HID, HID, HP, mlp2_w1, mlp2_b1, mlp2_w2, mlp2_b2, root2, bias2),
        (HID, HID // 2, HP2, mlp3_w1, mlp3_b1, mlp3_w2, mlp3_b2, root3, bias3),
    ]
    aggr = None
    for li, (nin, nout, op, w1, b1, w2, b2, root, bias) in enumerate(layers):
        w2pT, b2T = _prep_w2(w2, b2, nin, nout, op)
        rootp = _pad2(root, F, F)
        biasp = _pad2(bias.reshape(1, nout), 1, F)
        xj = _gather_rows(xl, src)
        msg = _edge_messages(eaT, xj, w1.T, b1.reshape(HID, 1), w2pT, b2T,
                             nin, op)
        aggr = _scatter_add(msg, dst3, zeros)
        if li < 2:
            xl = _node_update(xl, aggr, rootp, biasp)

    fc1p = _pad2(fc1_w, F, HP)
    fc1bp = _pad2(fc1_b.reshape(1, HID), 1, HP)
    outwp = _pad2(out_w, HP, F)
    outbp = _pad2(out_b.reshape(1, 1), 1, F)
    rootp3 = _pad2(root3, F, F)
    biasp3 = _pad2(bias3.reshape(1, HID // 2), 1, F)
    out = _node_pool_head(xl, aggr, rootp3, biasp3, batch_row,
                          fc1p, fc1bp, outwp, outbp)
    return out[:, :1]
